# baseline (speedup 1.0000x reference)
import functools

import jax
import jax.numpy as jnp
from jax import lax
from jax.experimental import pallas as pl
from jax.experimental.pallas import tpu as pltpu
from jax.experimental.pallas import tpu_sc as plsc

NUM_ENT = 50000
NUM_REL = 512
NUM_STEPS = 2
BSZ = 16
SEQ = 32
T_EDGES = 800000
DIM_HIDDEN = 768
H = 384
G3 = 3 * H

NC = 2
NS = 16
NW = NC * NS

ENTP = 50176
ZR = ENTP // NS
CROWS = ENTP // NW

EDGE_PAD = 819200
ROWS128 = EDGE_PAD // 128
ROWS_PER_W = ROWS128 // NW
CHUNK_ROWS = 5
NCHUNK = ROWS_PER_W // CHUNK_ROWS


EREAL = T_EDGES // 128
EPADR = ROWS128 - EREAL


def _dense_body(q_ref, qs_ref, wemb_ref,
                wih_ref, bih_ref, whh_ref, bhh_ref,
                w0_ref, b0_ref, w1_ref, b1_ref, wr_ref, br_ref,
                wh_ref, bh_ref,
                rel0_ref, rel1_ref, aux_ref,
                x_ref, gi_ref, hs_ref, gsem):
    f32 = jnp.float32
    qz = (q_ref[...] == 0).astype(f32)
    lens = (SEQ - jnp.sum(qz, axis=1, keepdims=True))

    LAG = 64

    def _row_copy(r):
        b = jnp.remainder(r, BSZ)
        t = r // BSZ
        idx = qs_ref[b, t]
        return pltpu.make_async_copy(wemb_ref.at[idx], x_ref.at[r], gsem)

    def _fire(r, carry):
        _row_copy(r).start()

        @pl.when(r >= LAG)
        def _():
            _row_copy(r - LAG).wait()
        return carry

    lax.fori_loop(0, SEQ * BSZ, _fire, 0)

    def _drain(r, carry):
        _row_copy(r).wait()
        return carry

    lax.fori_loop(SEQ * BSZ - LAG, SEQ * BSZ, _drain, 0)

    gi_ref[...] = (
        jnp.dot(x_ref[...], wih_ref[...], preferred_element_type=f32)
        + bih_ref[...]
    )

    def gru_dir(gi, h, gh):
        i_r = gi[:, 0:H]
        i_z = gi[:, H:2 * H]
        i_n = gi[:, 2 * H:3 * H]
        h_r = gh[:, 0:H]
        h_z = gh[:, H:2 * H]
        h_n = gh[:, 2 * H:3 * H]
        r = jax.nn.sigmoid(i_r + h_r)
        z = jax.nn.sigmoid(i_z + h_z)
        n = jnp.tanh(i_n + r * h_n)
        return (1.0 - z) * n + z * h

    def step(t, h_cat):
        h_f = h_cat[:, 0:H]
        h_b = h_cat[:, H:2 * H]
        gh = jnp.dot(h_cat, whh_ref[...], preferred_element_type=f32) + bhh_ref[...]
        gi_f = gi_ref[pl.ds(t * BSZ, BSZ), 0:G3]
        tb = SEQ - 1 - t
        gi_b = gi_ref[pl.ds(tb * BSZ, BSZ), G3:2 * G3]
        hf_new = gru_dir(gi_f, h_f, gh[:, 0:G3])
        hb_new = gru_dir(gi_b, h_b, gh[:, G3:2 * G3])
        mt_f = (t.astype(f32) < lens).astype(f32)
        mt_b = ((SEQ - 1 - t).astype(f32) < lens).astype(f32)
        h_f2 = mt_f * hf_new + (1.0 - mt_f) * h_f
        h_b2 = mt_b * hb_new + (1.0 - mt_b) * h_b
        hs_ref[pl.ds(t * BSZ, BSZ), 0:H] = h_f2
        hs_ref[pl.ds(tb * BSZ, BSZ), H:2 * H] = h_b2
        return jnp.concatenate([h_f2, h_b2], axis=1)

    h0 = jnp.zeros((BSZ, 2 * H), dtype=f32)
    hT = lax.fori_loop(0, SEQ, step, h0)
    q_emb = hT

    step_w = [w0_ref, w1_ref]
    step_b = [b0_ref, b1_ref]
    rel_refs = [rel0_ref, rel1_ref]
    ams = []
    for t in range(NUM_STEPS):
        cq = jnp.tanh(
            jnp.dot(q_emb, step_w[t][...], preferred_element_type=f32)
            + step_b[t][...]
        )
        cols = []
        for s in range(SEQ):
            blk = hs_ref[pl.ds(s * BSZ, BSZ), :]
            cols.append(jnp.sum(cq * blk, axis=1, keepdims=True))
        logits = jnp.concatenate(cols, axis=1)
        mx = jnp.max(logits, axis=1, keepdims=True)
        ex = jnp.exp(logits - mx)
        dist = ex / jnp.sum(ex, axis=1, keepdims=True)
        ctx = jnp.zeros((BSZ, 2 * H), dtype=f32)
        for s in range(SEQ):
            ctx = ctx + dist[:, s:s + 1] * hs_ref[pl.ds(s * BSZ, BSZ), :]
        rl = jnp.dot(ctx, wr_ref[...], preferred_element_type=f32) + br_ref[...]
        rmx = jnp.max(rl, axis=1, keepdims=True)
        rex = jnp.exp(rl - rmx)
        rel_refs[t][...] = rex / jnp.sum(rex, axis=1, keepdims=True)
        ii = lax.broadcasted_iota(jnp.int32, (BSZ, NUM_REL), 1)
        cand = jnp.where(rl >= rmx, ii, NUM_REL)
        ams.append(jnp.min(cand, axis=1, keepdims=True))

    prev_rel, curr_rel = ams[0], ams[1]
    cond = ((jnp.abs(prev_rel - curr_rel) == 1)
            & (jnp.remainder(jnp.minimum(prev_rel, curr_rel), 2) == 0))
    hop_logit = jnp.dot(q_emb, wh_ref[...], preferred_element_type=f32) + bh_ref[...]
    hmx = jnp.max(hop_logit, axis=1, keepdims=True)
    hex_ = jnp.exp(hop_logit - hmx)
    attn = hex_ / jnp.sum(hex_, axis=1, keepdims=True)
    a0 = attn[:, 0:1]
    a1 = attn[:, 1:2]
    m3 = (a1 > a0).astype(f32)
    aux_ref[:, 0:1] = cond.astype(f32)
    aux_ref[:, 1:2] = a0
    aux_ref[:, 2:3] = a1
    aux_ref[:, 3:4] = m3


def _dense_call(questions, word_emb,
                wih_cat, bih_cat, whh_cat, bhh_cat,
                w0, b0, w1, b1, wr, br, wh, bh):
    f32 = jnp.float32
    nin = 15
    specs = [pl.BlockSpec(memory_space=pltpu.VMEM) for _ in range(nin)]
    specs[0] = pl.BlockSpec(memory_space=pltpu.VMEM)
    specs[1] = pl.BlockSpec(memory_space=pltpu.SMEM)
    specs[2] = pl.BlockSpec(memory_space=pl.ANY)
    return pl.pallas_call(
        _dense_body,
        out_shape=(
            jax.ShapeDtypeStruct((BSZ, NUM_REL), f32),
            jax.ShapeDtypeStruct((BSZ, NUM_REL), f32),
            jax.ShapeDtypeStruct((BSZ, 4), f32),
        ),
        in_specs=specs,
        scratch_shapes=[
            pltpu.VMEM((SEQ * BSZ, 300), f32),
            pltpu.VMEM((SEQ * BSZ, 2 * G3), f32),
            pltpu.VMEM((SEQ * BSZ, 2 * H), f32),
            pltpu.SemaphoreType.DMA,
        ],
    )(questions, questions, word_emb,
      wih_cat, bih_cat, whh_cat, bhh_cat,
      w0, b0, w1, b1, wr, br, wh, bh)


_SC_MESH = plsc.VectorSubcoreMesh(
    core_axis_name="c", subcore_axis_name="s", num_cores=NC, num_subcores=NS)


def _hop_body(ent, relt, subj, reli, obj, zrows, out,
              subj_v, rel_v, obj_v, erow, rrow, sem_g, sem_s, sem_i, acc):
    c = lax.axis_index("c")
    s = lax.axis_index("s")
    wid = s * NC + c

    pltpu.sync_copy(zrows, acc.at[pl.ds(s * ZR, ZR)])
    plsc.subcore_barrier()

    def idx_cp(kk, i):
        row0 = wid * ROWS_PER_W + kk * CHUNK_ROWS
        return (
            pltpu.make_async_copy(subj.at[pl.ds(row0, CHUNK_ROWS)],
                                  subj_v.at[i], sem_i),
            pltpu.make_async_copy(reli.at[pl.ds(row0, CHUNK_ROWS)],
                                  rel_v.at[i], sem_i),
            pltpu.make_async_copy(obj.at[pl.ds(row0, CHUNK_ROWS)],
                                  obj_v.at[i], sem_i),
        )

    def fire_idx(kk, i):
        for cp in idx_cp(kk, i):
            cp.start()

    def wait_idx(kk, i):
        for cp in idx_cp(kk, i):
            cp.wait()

    def fire_gathers(b, i):
        for r in range(CHUNK_ROWS):
            pltpu.async_copy(ent.at[subj_v.at[i, r]],
                             erow.at[b, pl.ds(r * 128, 128)], sem_g)
            pltpu.async_copy(relt.at[rel_v.at[i, r]],
                             rrow.at[b, pl.ds(r * 128, 128)], sem_g)

    def wait_gathers(b, i):
        for r in range(CHUNK_ROWS):
            pltpu.make_async_copy(ent.at[subj_v.at[i, r]],
                                  erow.at[b, pl.ds(r * 128, 128)], sem_g).wait()
            pltpu.make_async_copy(relt.at[rel_v.at[i, r]],
                                  rrow.at[b, pl.ds(r * 128, 128)], sem_g).wait()

    def scatter(b, i):
        for r in range(CHUNK_ROWS):
            pltpu.async_copy(rrow.at[b, pl.ds(r * 128, 128)],
                             acc.at[obj_v.at[i, r]], sem_s, add=True)

    def drain_scatter(b, i):
        for r in range(CHUNK_ROWS):
            pltpu.make_async_copy(rrow.at[b, pl.ds(r * 128, 128)],
                                  acc.at[obj_v.at[i, r]], sem_s).wait()

    def mul(b):
        @pl.loop(0, CHUNK_ROWS * 128, unroll=8)
        def _mul(j):
            rrow[b, j, :] = rrow[b, j, :] * erow[b, j, :]

    fire_idx(0, 0)
    fire_idx(1, 1)
    wait_idx(0, 0)
    fire_gathers(0, 0)

    @pl.loop(0, NCHUNK - 4, step=6)
    def _six(k):
        for j in range(6):
            kk = k + j
            b = j % 3
            i = j
            i1 = (j + 1) % 6
            i2 = (j + 2) % 6
            iold = (j + 4) % 6
            nb = (b + 1) % 3

            @pl.when(kk >= 2)
            def _():
                drain_scatter(nb, iold)

            fire_idx(kk + 2, i2)
            wait_idx(kk + 1, i1)
            fire_gathers(nb, i1)
            wait_gathers(b, i)
            mul(b)
            scatter(b, i)

    for j in range(4):
        kk = NCHUNK - 4 + j
        b = j % 3
        i = j
        i1 = (j + 1) % 6
        i2 = (j + 2) % 6
        iold = (j + 4) % 6
        nb = (b + 1) % 3
        drain_scatter(nb, iold)
        if kk + 2 < NCHUNK:
            fire_idx(kk + 2, i2)
        if kk + 1 < NCHUNK:
            wait_idx(kk + 1, i1)
            fire_gathers(nb, i1)
        wait_gathers(b, i)
        mul(b)
        scatter(b, i)
    drain_scatter(2, 2)
    drain_scatter(0, 3)

    plsc.subcore_barrier()
    pltpu.sync_copy(acc.at[pl.ds(s * ZR, ZR)],
                    out.at[pl.ds(c * ENTP + s * ZR, ZR)])


_hop_call = functools.partial(
    pl.kernel,
    out_type=jax.ShapeDtypeStruct((NC * ENTP, 16), jnp.float32),
    mesh=_SC_MESH,
    compiler_params=pltpu.CompilerParams(
        use_tc_tiling_on_sc=False, needs_layout_passes=False),
    scratch_types=[
        pltpu.VMEM((6, CHUNK_ROWS, 128), jnp.int32),
        pltpu.VMEM((6, CHUNK_ROWS, 128), jnp.int32),
        pltpu.VMEM((6, CHUNK_ROWS, 128), jnp.int32),
        pltpu.VMEM((3, CHUNK_ROWS * 128, 16), jnp.float32),
        pltpu.VMEM((3, CHUNK_ROWS * 128, 16), jnp.float32),
        pltpu.SemaphoreType.DMA,
        pltpu.SemaphoreType.DMA,
        pltpu.SemaphoreType.DMA,
        pltpu.VMEM_SHARED((ENTP, 16), jnp.float32),
    ],
)(_hop_body)


def _combine0_body(part, out, p0, p1):
    c = lax.axis_index("c")
    s = lax.axis_index("s")
    wid = s * NC + c
    r0 = wid * CROWS
    pltpu.sync_copy(part.at[pl.ds(r0, CROWS)], p0)
    pltpu.sync_copy(part.at[pl.ds(ENTP + r0, CROWS)], p1)

    @pl.loop(0, CROWS, unroll=4)
    def _row(j):
        v = p0[j, :] + p1[j, :]
        p0[j, :] = v / jnp.maximum(v, 1.0)

    pltpu.sync_copy(p0, out.at[pl.ds(r0, CROWS)])


_combine0_call = functools.partial(
    pl.kernel,
    out_type=jax.ShapeDtypeStruct((ENTP, 16), jnp.float32),
    mesh=_SC_MESH,
    compiler_params=pltpu.CompilerParams(
        use_tc_tiling_on_sc=False, needs_layout_passes=False),
    scratch_types=[
        pltpu.VMEM((CROWS, 16), jnp.float32),
        pltpu.VMEM((CROWS, 16), jnp.float32),
    ],
)(_combine0_body)


def _final_body(part, e1, est, aux, out, p0, p1, e1v, esv, auxv, outt):
    c = lax.axis_index("c")
    s = lax.axis_index("s")
    wid = s * NC + c
    r0 = wid * CROWS
    pltpu.sync_copy(part.at[pl.ds(r0, CROWS)], p0)
    pltpu.sync_copy(part.at[pl.ds(ENTP + r0, CROWS)], p1)
    pltpu.sync_copy(e1.at[pl.ds(r0, CROWS)], e1v)
    pltpu.sync_copy(est.at[pl.ds(r0, CROWS)], esv)
    pltpu.sync_copy(aux, auxv)
    cond = auxv[0, :]
    a0 = auxv[1, :]
    a1 = auxv[2, :]
    m3 = auxv[3, :]

    @pl.loop(0, CROWS, unroll=4)
    def _row(j):
        v = p0[j, :] + p1[j, :]
        v = v / jnp.maximum(v, 1.0)
        es_row = esv[j, :]
        ent_m = cond * jnp.where(es_row > 0.9, 1.0, 0.0)
        v = (1.0 - ent_m) * v
        o = a0 * e1v[j, :] + a1 * v
        p0[j, :] = (1.0 - m3 * es_row) * o

    lanes = lax.iota(jnp.int32, 16)

    @pl.loop(0, CROWS // 16)
    def _tr(jv):
        rows = jv * 16 + lanes
        for b in range(BSZ):
            col = jnp.full((16,), b, jnp.int32)
            outt[b, pl.ds(jv * 16, 16)] = plsc.load_gather(p0, [rows, col])

    last = NUM_ENT - (NW - 1) * CROWS

    @pl.when(wid < NW - 1)
    def _():
        for b in range(BSZ):
            pltpu.sync_copy(outt.at[b], out.at[b, pl.ds(r0, CROWS)])

    @pl.when(wid == NW - 1)
    def _():
        for b in range(BSZ):
            pltpu.sync_copy(outt.at[b, pl.ds(0, last)],
                            out.at[b, pl.ds(r0, last)])


_final_call = functools.partial(
    pl.kernel,
    out_type=jax.ShapeDtypeStruct((BSZ, NUM_ENT), jnp.float32),
    mesh=_SC_MESH,
    compiler_params=pltpu.CompilerParams(
        use_tc_tiling_on_sc=False, needs_layout_passes=False),
    scratch_types=[
        pltpu.VMEM((CROWS, 16), jnp.float32),
        pltpu.VMEM((CROWS, 16), jnp.float32),
        pltpu.VMEM((CROWS, 16), jnp.float32),
        pltpu.VMEM((CROWS, 16), jnp.float32),
        pltpu.VMEM((4, 16), jnp.float32),
        pltpu.VMEM((BSZ, CROWS), jnp.float32),
    ],
)(_final_body)


def kernel(questions, e_s, subj_idx, rel_idx, obj_idx, word_emb,
           Wf_ih, Wf_hh, bf_ih, bf_hh, Wb_ih, Wb_hh, bb_ih, bb_hh,
           W_step0, b_step0, W_step1, b_step1, W_rel, b_rel, W_hop, b_hop):
    f32 = jnp.float32

    wih_cat = jnp.concatenate([Wf_ih, Wb_ih], axis=1)
    bih_cat = jnp.concatenate([bf_ih, bb_ih]).reshape(1, 2 * G3)
    zhh = jnp.zeros((H, G3), dtype=f32)
    whh_cat = jnp.concatenate([
        jnp.concatenate([Wf_hh, zhh], axis=1),
        jnp.concatenate([zhh, Wb_hh], axis=1),
    ], axis=0)
    bhh_cat = jnp.concatenate([bf_hh, bb_hh]).reshape(1, 2 * G3)

    rel0, rel1, aux = _dense_call(
        questions, word_emb,
        wih_cat, bih_cat, whh_cat, bhh_cat,
        W_step0, b_step0.reshape(1, -1), W_step1, b_step1.reshape(1, -1),
        W_rel, b_rel.reshape(1, -1), W_hop, b_hop.reshape(1, -1))

    es_t = jnp.pad(e_s.T, ((0, ENTP - NUM_ENT), (0, 0)))
    rel0_t = rel0.T
    rel1_t = rel1.T
    aux_t = aux.T

    npad = EDGE_PAD - T_EDGES
    subj_p = jnp.concatenate(
        [subj_idx, jnp.zeros((npad,), jnp.int32)]).reshape(ROWS128, 128)
    reli_p = jnp.concatenate(
        [rel_idx, jnp.zeros((npad,), jnp.int32)]).reshape(ROWS128, 128)
    dump = NUM_ENT + jnp.arange(npad, dtype=jnp.int32) % (ENTP - NUM_ENT)
    obj_p = jnp.concatenate([obj_idx, dump]).reshape(ROWS128, 128)

    zrows = jnp.zeros((ZR, 16), dtype=f32)

    part0 = _hop_call(es_t, rel0_t, subj_p, reli_p, obj_p, zrows)
    e1 = _combine0_call(part0)
    part1 = _hop_call(e1, rel1_t, subj_p, reli_p, obj_p, zrows)
    return _final_call(part1, e1, es_t, aux_t)


# speedup vs baseline: 1.0419x; 1.0419x over previous
import functools

import jax
import jax.numpy as jnp
from jax import lax
from jax.experimental import pallas as pl
from jax.experimental.pallas import tpu as pltpu
from jax.experimental.pallas import tpu_sc as plsc

NUM_ENT = 50000
NUM_REL = 512
NUM_STEPS = 2
BSZ = 16
SEQ = 32
T_EDGES = 800000
DIM_HIDDEN = 768
H = 384
G3 = 3 * H

NC = 2
NS = 16
NW = NC * NS

ENTP = 50176
ZR = ENTP // NS
CROWS = ENTP // NW

EDGE_PAD = 819200
ROWS128 = EDGE_PAD // 128
ROWS_PER_W = ROWS128 // NW
CHUNK_ROWS = 5
NCHUNK = ROWS_PER_W // CHUNK_ROWS


EREAL = T_EDGES // 128
EPADR = ROWS128 - EREAL


def _dense_body(q_ref, qs_ref, wemb_ref,
                wih_ref, bih_ref, whh_ref, bhh_ref,
                w0_ref, b0_ref, w1_ref, b1_ref, wr_ref, br_ref,
                wh_ref, bh_ref,
                rel0_ref, rel1_ref, aux_ref,
                x_ref, gi_ref, hs_ref, gsem):
    f32 = jnp.float32
    qz = (q_ref[...] == 0).astype(f32)
    lens = (SEQ - jnp.sum(qz, axis=1, keepdims=True))

    LAG = 64

    def _row_copy(r):
        b = jnp.remainder(r, BSZ)
        t = r // BSZ
        idx = qs_ref[b, t]
        return pltpu.make_async_copy(wemb_ref.at[idx], x_ref.at[r], gsem)

    def _fire(r, carry):
        _row_copy(r).start()

        @pl.when(r >= LAG)
        def _():
            _row_copy(r - LAG).wait()
        return carry

    lax.fori_loop(0, SEQ * BSZ, _fire, 0)

    def _drain(r, carry):
        _row_copy(r).wait()
        return carry

    lax.fori_loop(SEQ * BSZ - LAG, SEQ * BSZ, _drain, 0)

    gi_ref[...] = (
        jnp.dot(x_ref[...], wih_ref[...], preferred_element_type=f32)
        + bih_ref[...]
    )

    def gru_dir(gi, h, gh):
        i_r = gi[:, 0:H]
        i_z = gi[:, H:2 * H]
        i_n = gi[:, 2 * H:3 * H]
        h_r = gh[:, 0:H]
        h_z = gh[:, H:2 * H]
        h_n = gh[:, 2 * H:3 * H]
        r = jax.nn.sigmoid(i_r + h_r)
        z = jax.nn.sigmoid(i_z + h_z)
        n = jnp.tanh(i_n + r * h_n)
        return (1.0 - z) * n + z * h

    def step(t, h_cat):
        h_f = h_cat[:, 0:H]
        h_b = h_cat[:, H:2 * H]
        gh = jnp.dot(h_cat, whh_ref[...], preferred_element_type=f32) + bhh_ref[...]
        gi_f = gi_ref[pl.ds(t * BSZ, BSZ), 0:G3]
        tb = SEQ - 1 - t
        gi_b = gi_ref[pl.ds(tb * BSZ, BSZ), G3:2 * G3]
        hf_new = gru_dir(gi_f, h_f, gh[:, 0:G3])
        hb_new = gru_dir(gi_b, h_b, gh[:, G3:2 * G3])
        mt_f = (t.astype(f32) < lens).astype(f32)
        mt_b = ((SEQ - 1 - t).astype(f32) < lens).astype(f32)
        h_f2 = mt_f * hf_new + (1.0 - mt_f) * h_f
        h_b2 = mt_b * hb_new + (1.0 - mt_b) * h_b
        hs_ref[pl.ds(t * BSZ, BSZ), 0:H] = h_f2
        hs_ref[pl.ds(tb * BSZ, BSZ), H:2 * H] = h_b2
        return jnp.concatenate([h_f2, h_b2], axis=1)

    h0 = jnp.zeros((BSZ, 2 * H), dtype=f32)
    hT = lax.fori_loop(0, SEQ, step, h0)
    q_emb = hT

    step_w = [w0_ref, w1_ref]
    step_b = [b0_ref, b1_ref]
    rel_refs = [rel0_ref, rel1_ref]
    ams = []
    for t in range(NUM_STEPS):
        cq = jnp.tanh(
            jnp.dot(q_emb, step_w[t][...], preferred_element_type=f32)
            + step_b[t][...]
        )
        cols = []
        for s in range(SEQ):
            blk = hs_ref[pl.ds(s * BSZ, BSZ), :]
            cols.append(jnp.sum(cq * blk, axis=1, keepdims=True))
        logits = jnp.concatenate(cols, axis=1)
        mx = jnp.max(logits, axis=1, keepdims=True)
        ex = jnp.exp(logits - mx)
        dist = ex / jnp.sum(ex, axis=1, keepdims=True)
        ctx = jnp.zeros((BSZ, 2 * H), dtype=f32)
        for s in range(SEQ):
            ctx = ctx + dist[:, s:s + 1] * hs_ref[pl.ds(s * BSZ, BSZ), :]
        rl = jnp.dot(ctx, wr_ref[...], preferred_element_type=f32) + br_ref[...]
        rmx = jnp.max(rl, axis=1, keepdims=True)
        rex = jnp.exp(rl - rmx)
        rel_refs[t][...] = rex / jnp.sum(rex, axis=1, keepdims=True)
        ii = lax.broadcasted_iota(jnp.int32, (BSZ, NUM_REL), 1)
        cand = jnp.where(rl >= rmx, ii, NUM_REL)
        ams.append(jnp.min(cand, axis=1, keepdims=True))

    prev_rel, curr_rel = ams[0], ams[1]
    cond = ((jnp.abs(prev_rel - curr_rel) == 1)
            & (jnp.remainder(jnp.minimum(prev_rel, curr_rel), 2) == 0))
    hop_logit = jnp.dot(q_emb, wh_ref[...], preferred_element_type=f32) + bh_ref[...]
    hmx = jnp.max(hop_logit, axis=1, keepdims=True)
    hex_ = jnp.exp(hop_logit - hmx)
    attn = hex_ / jnp.sum(hex_, axis=1, keepdims=True)
    a0 = attn[:, 0:1]
    a1 = attn[:, 1:2]
    m3 = (a1 > a0).astype(f32)
    aux_ref[:, 0:1] = cond.astype(f32)
    aux_ref[:, 1:2] = a0
    aux_ref[:, 2:3] = a1
    aux_ref[:, 3:4] = m3


def _dense_call(questions, word_emb,
                wih_cat, bih_cat, whh_cat, bhh_cat,
                w0, b0, w1, b1, wr, br, wh, bh):
    f32 = jnp.float32
    nin = 15
    specs = [pl.BlockSpec(memory_space=pltpu.VMEM) for _ in range(nin)]
    specs[0] = pl.BlockSpec(memory_space=pltpu.VMEM)
    specs[1] = pl.BlockSpec(memory_space=pltpu.SMEM)
    specs[2] = pl.BlockSpec(memory_space=pl.ANY)
    return pl.pallas_call(
        _dense_body,
        out_shape=(
            jax.ShapeDtypeStruct((BSZ, NUM_REL), f32),
            jax.ShapeDtypeStruct((BSZ, NUM_REL), f32),
            jax.ShapeDtypeStruct((BSZ, 4), f32),
        ),
        in_specs=specs,
        scratch_shapes=[
            pltpu.VMEM((SEQ * BSZ, 300), f32),
            pltpu.VMEM((SEQ * BSZ, 2 * G3), f32),
            pltpu.VMEM((SEQ * BSZ, 2 * H), f32),
            pltpu.SemaphoreType.DMA,
        ],
    )(questions, questions, word_emb,
      wih_cat, bih_cat, whh_cat, bhh_cat,
      w0, b0, w1, b1, wr, br, wh, bh)


_SC_MESH = plsc.VectorSubcoreMesh(
    core_axis_name="c", subcore_axis_name="s", num_cores=NC, num_subcores=NS)


def _hop_body(ent, relt, subj, reli, obj, zrows, out,
              subj_v, rel_v, obj_v, erow, rrow, sem_g, sem_s, acc):
    c = lax.axis_index("c")
    s = lax.axis_index("s")
    wid = s * NC + c

    pltpu.sync_copy(zrows, acc.at[pl.ds(s * ZR, ZR)])
    plsc.subcore_barrier()

    def fire(kk, b):
        row0 = wid * ROWS_PER_W + kk * CHUNK_ROWS
        pltpu.sync_copy(subj.at[pl.ds(row0, CHUNK_ROWS)], subj_v.at[b])
        pltpu.sync_copy(reli.at[pl.ds(row0, CHUNK_ROWS)], rel_v.at[b])
        pltpu.sync_copy(obj.at[pl.ds(row0, CHUNK_ROWS)], obj_v.at[b])
        for r in range(CHUNK_ROWS):
            pltpu.async_copy(ent.at[subj_v.at[b, r]],
                             erow.at[b, pl.ds(r * 128, 128)], sem_g)
            pltpu.async_copy(relt.at[rel_v.at[b, r]],
                             rrow.at[b, pl.ds(r * 128, 128)], sem_g)

    def wait_gathers(b):
        for r in range(CHUNK_ROWS):
            pltpu.make_async_copy(ent.at[subj_v.at[b, r]],
                                  erow.at[b, pl.ds(r * 128, 128)], sem_g).wait()
            pltpu.make_async_copy(relt.at[rel_v.at[b, r]],
                                  rrow.at[b, pl.ds(r * 128, 128)], sem_g).wait()

    def scatter(b):
        for r in range(CHUNK_ROWS):
            pltpu.async_copy(rrow.at[b, pl.ds(r * 128, 128)],
                             acc.at[obj_v.at[b, r]], sem_s, add=True)

    def drain_scatter(b):
        for r in range(CHUNK_ROWS):
            pltpu.make_async_copy(rrow.at[b, pl.ds(r * 128, 128)],
                                  acc.at[obj_v.at[b, r]], sem_s).wait()

    def mul(b):
        @pl.loop(0, CHUNK_ROWS * 128, unroll=8)
        def _mul(j):
            rrow[b, j, :] = rrow[b, j, :] * erow[b, j, :]

    fire(0, 0)

    @pl.loop(0, NCHUNK - 1, step=3)
    def _triple(k):
        for b in range(3):
            kk = k + b
            nb = (b + 1) % 3

            @pl.when(kk >= 2)
            def _():
                drain_scatter(nb)

            fire(kk + 1, nb)
            wait_gathers(b)
            mul(b)
            scatter(b)

    drain_scatter(1)
    wait_gathers(0)
    mul(0)
    scatter(0)
    drain_scatter(2)
    drain_scatter(0)

    plsc.subcore_barrier()
    pltpu.sync_copy(acc.at[pl.ds(s * ZR, ZR)],
                    out.at[pl.ds(c * ENTP + s * ZR, ZR)])


_hop_call = functools.partial(
    pl.kernel,
    out_type=jax.ShapeDtypeStruct((NC * ENTP, 16), jnp.float32),
    mesh=_SC_MESH,
    compiler_params=pltpu.CompilerParams(
        use_tc_tiling_on_sc=False, needs_layout_passes=False),
    scratch_types=[
        pltpu.VMEM((3, CHUNK_ROWS, 128), jnp.int32),
        pltpu.VMEM((3, CHUNK_ROWS, 128), jnp.int32),
        pltpu.VMEM((3, CHUNK_ROWS, 128), jnp.int32),
        pltpu.VMEM((3, CHUNK_ROWS * 128, 16), jnp.float32),
        pltpu.VMEM((3, CHUNK_ROWS * 128, 16), jnp.float32),
        pltpu.SemaphoreType.DMA,
        pltpu.SemaphoreType.DMA,
        pltpu.VMEM_SHARED((ENTP, 16), jnp.float32),
    ],
)(_hop_body)


def _combine0_body(part, out, p0, p1):
    c = lax.axis_index("c")
    s = lax.axis_index("s")
    wid = s * NC + c
    r0 = wid * CROWS
    pltpu.sync_copy(part.at[pl.ds(r0, CROWS)], p0)
    pltpu.sync_copy(part.at[pl.ds(ENTP + r0, CROWS)], p1)

    @pl.loop(0, CROWS, unroll=4)
    def _row(j):
        v = p0[j, :] + p1[j, :]
        p0[j, :] = v / jnp.maximum(v, 1.0)

    pltpu.sync_copy(p0, out.at[pl.ds(r0, CROWS)])


_combine0_call = functools.partial(
    pl.kernel,
    out_type=jax.ShapeDtypeStruct((ENTP, 16), jnp.float32),
    mesh=_SC_MESH,
    compiler_params=pltpu.CompilerParams(
        use_tc_tiling_on_sc=False, needs_layout_passes=False),
    scratch_types=[
        pltpu.VMEM((CROWS, 16), jnp.float32),
        pltpu.VMEM((CROWS, 16), jnp.float32),
    ],
)(_combine0_body)


def _final_body(part, e1, est, aux, out, p0, p1, e1v, esv, auxv, outt):
    c = lax.axis_index("c")
    s = lax.axis_index("s")
    wid = s * NC + c
    r0 = wid * CROWS
    pltpu.sync_copy(part.at[pl.ds(r0, CROWS)], p0)
    pltpu.sync_copy(part.at[pl.ds(ENTP + r0, CROWS)], p1)
    pltpu.sync_copy(e1.at[pl.ds(r0, CROWS)], e1v)
    esr = NUM_ENT - (NW - 1) * CROWS

    @pl.when(wid < NW - 1)
    def _():
        pltpu.sync_copy(est.at[pl.ds(r0, CROWS)], esv)

    @pl.when(wid == NW - 1)
    def _():
        pltpu.sync_copy(est.at[pl.ds(r0, esr)], esv.at[pl.ds(0, esr)])

    pltpu.sync_copy(aux, auxv)
    cond = auxv[0, :]
    a0 = auxv[1, :]
    a1 = auxv[2, :]
    m3 = auxv[3, :]

    @pl.loop(0, CROWS, unroll=4)
    def _row(j):
        v = p0[j, :] + p1[j, :]
        v = v / jnp.maximum(v, 1.0)
        es_row = esv[j, :]
        ent_m = cond * jnp.where(es_row > 0.9, 1.0, 0.0)
        v = (1.0 - ent_m) * v
        o = a0 * e1v[j, :] + a1 * v
        p0[j, :] = (1.0 - m3 * es_row) * o

    lanes = lax.iota(jnp.int32, 16)

    @pl.loop(0, CROWS // 16)
    def _tr(jv):
        rows = jv * 16 + lanes
        for b in range(BSZ):
            col = jnp.full((16,), b, jnp.int32)
            outt[b, pl.ds(jv * 16, 16)] = plsc.load_gather(p0, [rows, col])

    last = NUM_ENT - (NW - 1) * CROWS

    @pl.when(wid < NW - 1)
    def _():
        for b in range(BSZ):
            pltpu.sync_copy(outt.at[b], out.at[b, pl.ds(r0, CROWS)])

    @pl.when(wid == NW - 1)
    def _():
        for b in range(BSZ):
            pltpu.sync_copy(outt.at[b, pl.ds(0, last)],
                            out.at[b, pl.ds(r0, last)])


_final_call = functools.partial(
    pl.kernel,
    out_type=jax.ShapeDtypeStruct((BSZ, NUM_ENT), jnp.float32),
    mesh=_SC_MESH,
    compiler_params=pltpu.CompilerParams(
        use_tc_tiling_on_sc=False, needs_layout_passes=False),
    scratch_types=[
        pltpu.VMEM((CROWS, 16), jnp.float32),
        pltpu.VMEM((CROWS, 16), jnp.float32),
        pltpu.VMEM((CROWS, 16), jnp.float32),
        pltpu.VMEM((CROWS, 16), jnp.float32),
        pltpu.VMEM((4, 16), jnp.float32),
        pltpu.VMEM((BSZ, CROWS), jnp.float32),
    ],
)(_final_body)


def kernel(questions, e_s, subj_idx, rel_idx, obj_idx, word_emb,
           Wf_ih, Wf_hh, bf_ih, bf_hh, Wb_ih, Wb_hh, bb_ih, bb_hh,
           W_step0, b_step0, W_step1, b_step1, W_rel, b_rel, W_hop, b_hop):
    f32 = jnp.float32

    wih_cat = jnp.concatenate([Wf_ih, Wb_ih], axis=1)
    bih_cat = jnp.concatenate([bf_ih, bb_ih]).reshape(1, 2 * G3)
    zhh = jnp.zeros((H, G3), dtype=f32)
    whh_cat = jnp.concatenate([
        jnp.concatenate([Wf_hh, zhh], axis=1),
        jnp.concatenate([zhh, Wb_hh], axis=1),
    ], axis=0)
    bhh_cat = jnp.concatenate([bf_hh, bb_hh]).reshape(1, 2 * G3)

    rel0, rel1, aux = _dense_call(
        questions, word_emb,
        wih_cat, bih_cat, whh_cat, bhh_cat,
        W_step0, b_step0.reshape(1, -1), W_step1, b_step1.reshape(1, -1),
        W_rel, b_rel.reshape(1, -1), W_hop, b_hop.reshape(1, -1))

    es_t = e_s.T
    rel0_t = rel0.T
    rel1_t = rel1.T
    aux_t = aux.T

    npad = EDGE_PAD - T_EDGES
    subj_p = jnp.concatenate(
        [subj_idx, jnp.zeros((npad,), jnp.int32)]).reshape(ROWS128, 128)
    reli_p = jnp.concatenate(
        [rel_idx, jnp.zeros((npad,), jnp.int32)]).reshape(ROWS128, 128)
    dump = NUM_ENT + jnp.arange(npad, dtype=jnp.int32) % (ENTP - NUM_ENT)
    obj_p = jnp.concatenate([obj_idx, dump]).reshape(ROWS128, 128)

    zrows = jnp.zeros((ZR, 16), dtype=f32)

    part0 = _hop_call(es_t, rel0_t, subj_p, reli_p, obj_p, zrows)
    e1 = _combine0_call(part0)
    part1 = _hop_call(e1, rel1_t, subj_p, reli_p, obj_p, zrows)
    return _final_call(part1, e1, es_t, aux_t)


# speedup vs baseline: 1.0570x; 1.0145x over previous
import functools

import jax
import jax.numpy as jnp
from jax import lax
from jax.experimental import pallas as pl
from jax.experimental.pallas import tpu as pltpu
from jax.experimental.pallas import tpu_sc as plsc

NUM_ENT = 50000
NUM_REL = 512
NUM_STEPS = 2
BSZ = 16
SEQ = 32
T_EDGES = 800000
DIM_HIDDEN = 768
H = 384
G3 = 3 * H

NC = 2
NS = 16
NW = NC * NS

ENTP = 50176
ZR = ENTP // NS
CROWS = ENTP // NW

EDGE_PAD = 819200
ROWS128 = EDGE_PAD // 128
ROWS_PER_W = ROWS128 // NW
CHUNK_ROWS = 4
NCHUNK = ROWS_PER_W // CHUNK_ROWS


EREAL = T_EDGES // 128
EPADR = ROWS128 - EREAL


def _dense_body(q_ref, qs_ref, wemb_ref,
                wih_ref, bih_ref, whh_ref, bhh_ref,
                w0_ref, b0_ref, w1_ref, b1_ref, wr_ref, br_ref,
                wh_ref, bh_ref,
                rel0_ref, rel1_ref, aux_ref,
                x_ref, gi_ref, hs_ref, gsem):
    f32 = jnp.float32
    qz = (q_ref[...] == 0).astype(f32)
    lens = (SEQ - jnp.sum(qz, axis=1, keepdims=True))

    LAG = 64

    def _row_copy(r):
        b = jnp.remainder(r, BSZ)
        t = r // BSZ
        idx = qs_ref[b, t]
        return pltpu.make_async_copy(wemb_ref.at[idx], x_ref.at[r], gsem)

    def _fire(r, carry):
        _row_copy(r).start()

        @pl.when(r >= LAG)
        def _():
            _row_copy(r - LAG).wait()
        return carry

    lax.fori_loop(0, SEQ * BSZ, _fire, 0)

    def _drain(r, carry):
        _row_copy(r).wait()
        return carry

    lax.fori_loop(SEQ * BSZ - LAG, SEQ * BSZ, _drain, 0)

    gi_ref[...] = (
        jnp.dot(x_ref[...], wih_ref[...], preferred_element_type=f32)
        + bih_ref[...]
    )

    def gru_dir(gi, h, gh):
        i_r = gi[:, 0:H]
        i_z = gi[:, H:2 * H]
        i_n = gi[:, 2 * H:3 * H]
        h_r = gh[:, 0:H]
        h_z = gh[:, H:2 * H]
        h_n = gh[:, 2 * H:3 * H]
        r = jax.nn.sigmoid(i_r + h_r)
        z = jax.nn.sigmoid(i_z + h_z)
        n = jnp.tanh(i_n + r * h_n)
        return (1.0 - z) * n + z * h

    def step(t, h_cat):
        h_f = h_cat[:, 0:H]
        h_b = h_cat[:, H:2 * H]
        gh = jnp.dot(h_cat, whh_ref[...], preferred_element_type=f32) + bhh_ref[...]
        gi_f = gi_ref[pl.ds(t * BSZ, BSZ), 0:G3]
        tb = SEQ - 1 - t
        gi_b = gi_ref[pl.ds(tb * BSZ, BSZ), G3:2 * G3]
        hf_new = gru_dir(gi_f, h_f, gh[:, 0:G3])
        hb_new = gru_dir(gi_b, h_b, gh[:, G3:2 * G3])
        mt_f = (t.astype(f32) < lens).astype(f32)
        mt_b = ((SEQ - 1 - t).astype(f32) < lens).astype(f32)
        h_f2 = mt_f * hf_new + (1.0 - mt_f) * h_f
        h_b2 = mt_b * hb_new + (1.0 - mt_b) * h_b
        hs_ref[pl.ds(t * BSZ, BSZ), 0:H] = h_f2
        hs_ref[pl.ds(tb * BSZ, BSZ), H:2 * H] = h_b2
        return jnp.concatenate([h_f2, h_b2], axis=1)

    h0 = jnp.zeros((BSZ, 2 * H), dtype=f32)
    hT = lax.fori_loop(0, SEQ, step, h0)
    q_emb = hT

    step_w = [w0_ref, w1_ref]
    step_b = [b0_ref, b1_ref]
    rel_refs = [rel0_ref, rel1_ref]
    ams = []
    for t in range(NUM_STEPS):
        cq = jnp.tanh(
            jnp.dot(q_emb, step_w[t][...], preferred_element_type=f32)
            + step_b[t][...]
        )
        cols = []
        for s in range(SEQ):
            blk = hs_ref[pl.ds(s * BSZ, BSZ), :]
            cols.append(jnp.sum(cq * blk, axis=1, keepdims=True))
        logits = jnp.concatenate(cols, axis=1)
        mx = jnp.max(logits, axis=1, keepdims=True)
        ex = jnp.exp(logits - mx)
        dist = ex / jnp.sum(ex, axis=1, keepdims=True)
        ctx = jnp.zeros((BSZ, 2 * H), dtype=f32)
        for s in range(SEQ):
            ctx = ctx + dist[:, s:s + 1] * hs_ref[pl.ds(s * BSZ, BSZ), :]
        rl = jnp.dot(ctx, wr_ref[...], preferred_element_type=f32) + br_ref[...]
        rmx = jnp.max(rl, axis=1, keepdims=True)
        rex = jnp.exp(rl - rmx)
        rel_refs[t][...] = rex / jnp.sum(rex, axis=1, keepdims=True)
        ii = lax.broadcasted_iota(jnp.int32, (BSZ, NUM_REL), 1)
        cand = jnp.where(rl >= rmx, ii, NUM_REL)
        ams.append(jnp.min(cand, axis=1, keepdims=True))

    prev_rel, curr_rel = ams[0], ams[1]
    cond = ((jnp.abs(prev_rel - curr_rel) == 1)
            & (jnp.remainder(jnp.minimum(prev_rel, curr_rel), 2) == 0))
    hop_logit = jnp.dot(q_emb, wh_ref[...], preferred_element_type=f32) + bh_ref[...]
    hmx = jnp.max(hop_logit, axis=1, keepdims=True)
    hex_ = jnp.exp(hop_logit - hmx)
    attn = hex_ / jnp.sum(hex_, axis=1, keepdims=True)
    a0 = attn[:, 0:1]
    a1 = attn[:, 1:2]
    m3 = (a1 > a0).astype(f32)
    aux_ref[:, 0:1] = cond.astype(f32)
    aux_ref[:, 1:2] = a0
    aux_ref[:, 2:3] = a1
    aux_ref[:, 3:4] = m3


def _dense_call(questions, word_emb,
                wih_cat, bih_cat, whh_cat, bhh_cat,
                w0, b0, w1, b1, wr, br, wh, bh):
    f32 = jnp.float32
    nin = 15
    specs = [pl.BlockSpec(memory_space=pltpu.VMEM) for _ in range(nin)]
    specs[0] = pl.BlockSpec(memory_space=pltpu.VMEM)
    specs[1] = pl.BlockSpec(memory_space=pltpu.SMEM)
    specs[2] = pl.BlockSpec(memory_space=pl.ANY)
    return pl.pallas_call(
        _dense_body,
        out_shape=(
            jax.ShapeDtypeStruct((BSZ, NUM_REL), f32),
            jax.ShapeDtypeStruct((BSZ, NUM_REL), f32),
            jax.ShapeDtypeStruct((BSZ, 4), f32),
        ),
        in_specs=specs,
        scratch_shapes=[
            pltpu.VMEM((SEQ * BSZ, 300), f32),
            pltpu.VMEM((SEQ * BSZ, 2 * G3), f32),
            pltpu.VMEM((SEQ * BSZ, 2 * H), f32),
            pltpu.SemaphoreType.DMA,
        ],
    )(questions, questions, word_emb,
      wih_cat, bih_cat, whh_cat, bhh_cat,
      w0, b0, w1, b1, wr, br, wh, bh)


_SC_MESH = plsc.VectorSubcoreMesh(
    core_axis_name="c", subcore_axis_name="s", num_cores=NC, num_subcores=NS)


def _hop_body(ent, relt, subj, reli, obj, zrows, out,
              subj_v, rel_v, obj_v, erow, rrow, sem_g, sem_s, acc):
    c = lax.axis_index("c")
    s = lax.axis_index("s")
    wid = s * NC + c

    pltpu.sync_copy(zrows, acc.at[pl.ds(s * ZR, ZR)])
    plsc.subcore_barrier()

    def fire(kk, b):
        row0 = wid * ROWS_PER_W + kk * CHUNK_ROWS
        pltpu.sync_copy(subj.at[pl.ds(row0, CHUNK_ROWS)], subj_v.at[b])
        pltpu.sync_copy(reli.at[pl.ds(row0, CHUNK_ROWS)], rel_v.at[b])
        pltpu.sync_copy(obj.at[pl.ds(row0, CHUNK_ROWS)], obj_v.at[b])
        for r in range(CHUNK_ROWS):
            pltpu.async_copy(ent.at[subj_v.at[b, r]],
                             erow.at[b, pl.ds(r * 128, 128)], sem_g)
            pltpu.async_copy(relt.at[rel_v.at[b, r]],
                             rrow.at[b, pl.ds(r * 128, 128)], sem_g)

    def wait_gathers(b):
        for r in range(CHUNK_ROWS):
            pltpu.make_async_copy(ent.at[subj_v.at[b, r]],
                                  erow.at[b, pl.ds(r * 128, 128)], sem_g).wait()
            pltpu.make_async_copy(relt.at[rel_v.at[b, r]],
                                  rrow.at[b, pl.ds(r * 128, 128)], sem_g).wait()

    def scatter(b):
        for r in range(CHUNK_ROWS):
            pltpu.async_copy(rrow.at[b, pl.ds(r * 128, 128)],
                             acc.at[obj_v.at[b, r]], sem_s, add=True)

    def drain_scatter(b):
        for r in range(CHUNK_ROWS):
            pltpu.make_async_copy(rrow.at[b, pl.ds(r * 128, 128)],
                                  acc.at[obj_v.at[b, r]], sem_s).wait()

    def mul(b):
        @pl.loop(0, CHUNK_ROWS * 128, unroll=8)
        def _mul(j):
            rrow[b, j, :] = rrow[b, j, :] * erow[b, j, :]

    fire(0, 0)
    fire(1, 1)

    @pl.loop(0, NCHUNK - 2, step=4)
    def _quad(k):
        for j in range(4):
            kk = k + j
            b = j
            n2 = (j + 2) % 4

            @pl.when(kk >= 2)
            def _():
                drain_scatter(n2)

            fire(kk + 2, n2)
            wait_gathers(b)
            mul(b)
            scatter(b)

    drain_scatter(2)
    wait_gathers(0)
    mul(0)
    scatter(0)
    drain_scatter(3)
    wait_gathers(1)
    mul(1)
    scatter(1)
    drain_scatter(0)
    drain_scatter(1)

    plsc.subcore_barrier()
    pltpu.sync_copy(acc.at[pl.ds(s * ZR, ZR)],
                    out.at[pl.ds(c * ENTP + s * ZR, ZR)])


_hop_call = functools.partial(
    pl.kernel,
    out_type=jax.ShapeDtypeStruct((NC * ENTP, 16), jnp.float32),
    mesh=_SC_MESH,
    compiler_params=pltpu.CompilerParams(
        use_tc_tiling_on_sc=False, needs_layout_passes=False),
    scratch_types=[
        pltpu.VMEM((4, CHUNK_ROWS, 128), jnp.int32),
        pltpu.VMEM((4, CHUNK_ROWS, 128), jnp.int32),
        pltpu.VMEM((4, CHUNK_ROWS, 128), jnp.int32),
        pltpu.VMEM((4, CHUNK_ROWS * 128, 16), jnp.float32),
        pltpu.VMEM((4, CHUNK_ROWS * 128, 16), jnp.float32),
        pltpu.SemaphoreType.DMA,
        pltpu.SemaphoreType.DMA,
        pltpu.VMEM_SHARED((ENTP, 16), jnp.float32),
    ],
)(_hop_body)


def _combine0_body(part, out, p0, p1):
    c = lax.axis_index("c")
    s = lax.axis_index("s")
    wid = s * NC + c
    r0 = wid * CROWS
    pltpu.sync_copy(part.at[pl.ds(r0, CROWS)], p0)
    pltpu.sync_copy(part.at[pl.ds(ENTP + r0, CROWS)], p1)

    @pl.loop(0, CROWS, unroll=4)
    def _row(j):
        v = p0[j, :] + p1[j, :]
        p0[j, :] = v / jnp.maximum(v, 1.0)

    pltpu.sync_copy(p0, out.at[pl.ds(r0, CROWS)])


_combine0_call = functools.partial(
    pl.kernel,
    out_type=jax.ShapeDtypeStruct((ENTP, 16), jnp.float32),
    mesh=_SC_MESH,
    compiler_params=pltpu.CompilerParams(
        use_tc_tiling_on_sc=False, needs_layout_passes=False),
    scratch_types=[
        pltpu.VMEM((CROWS, 16), jnp.float32),
        pltpu.VMEM((CROWS, 16), jnp.float32),
    ],
)(_combine0_body)


def _final_body(part, e1, est, aux, out, p0, p1, e1v, esv, auxv, outt):
    c = lax.axis_index("c")
    s = lax.axis_index("s")
    wid = s * NC + c
    r0 = wid * CROWS
    pltpu.sync_copy(part.at[pl.ds(r0, CROWS)], p0)
    pltpu.sync_copy(part.at[pl.ds(ENTP + r0, CROWS)], p1)
    pltpu.sync_copy(e1.at[pl.ds(r0, CROWS)], e1v)
    esr = NUM_ENT - (NW - 1) * CROWS

    @pl.when(wid < NW - 1)
    def _():
        pltpu.sync_copy(est.at[pl.ds(r0, CROWS)], esv)

    @pl.when(wid == NW - 1)
    def _():
        pltpu.sync_copy(est.at[pl.ds(r0, esr)], esv.at[pl.ds(0, esr)])

    pltpu.sync_copy(aux, auxv)
    cond = auxv[0, :]
    a0 = auxv[1, :]
    a1 = auxv[2, :]
    m3 = auxv[3, :]

    @pl.loop(0, CROWS, unroll=4)
    def _row(j):
        v = p0[j, :] + p1[j, :]
        v = v / jnp.maximum(v, 1.0)
        es_row = esv[j, :]
        ent_m = cond * jnp.where(es_row > 0.9, 1.0, 0.0)
        v = (1.0 - ent_m) * v
        o = a0 * e1v[j, :] + a1 * v
        p0[j, :] = (1.0 - m3 * es_row) * o

    lanes = lax.iota(jnp.int32, 16)

    @pl.loop(0, CROWS // 16)
    def _tr(jv):
        rows = jv * 16 + lanes
        for b in range(BSZ):
            col = jnp.full((16,), b, jnp.int32)
            outt[b, pl.ds(jv * 16, 16)] = plsc.load_gather(p0, [rows, col])

    last = NUM_ENT - (NW - 1) * CROWS

    @pl.when(wid < NW - 1)
    def _():
        for b in range(BSZ):
            pltpu.sync_copy(outt.at[b], out.at[b, pl.ds(r0, CROWS)])

    @pl.when(wid == NW - 1)
    def _():
        for b in range(BSZ):
            pltpu.sync_copy(outt.at[b, pl.ds(0, last)],
                            out.at[b, pl.ds(r0, last)])


_final_call = functools.partial(
    pl.kernel,
    out_type=jax.ShapeDtypeStruct((BSZ, NUM_ENT), jnp.float32),
    mesh=_SC_MESH,
    compiler_params=pltpu.CompilerParams(
        use_tc_tiling_on_sc=False, needs_layout_passes=False),
    scratch_types=[
        pltpu.VMEM((CROWS, 16), jnp.float32),
        pltpu.VMEM((CROWS, 16), jnp.float32),
        pltpu.VMEM((CROWS, 16), jnp.float32),
        pltpu.VMEM((CROWS, 16), jnp.float32),
        pltpu.VMEM((4, 16), jnp.float32),
        pltpu.VMEM((BSZ, CROWS), jnp.float32),
    ],
)(_final_body)


def kernel(questions, e_s, subj_idx, rel_idx, obj_idx, word_emb,
           Wf_ih, Wf_hh, bf_ih, bf_hh, Wb_ih, Wb_hh, bb_ih, bb_hh,
           W_step0, b_step0, W_step1, b_step1, W_rel, b_rel, W_hop, b_hop):
    f32 = jnp.float32

    wih_cat = jnp.concatenate([Wf_ih, Wb_ih], axis=1)
    bih_cat = jnp.concatenate([bf_ih, bb_ih]).reshape(1, 2 * G3)
    zhh = jnp.zeros((H, G3), dtype=f32)
    whh_cat = jnp.concatenate([
        jnp.concatenate([Wf_hh, zhh], axis=1),
        jnp.concatenate([zhh, Wb_hh], axis=1),
    ], axis=0)
    bhh_cat = jnp.concatenate([bf_hh, bb_hh]).reshape(1, 2 * G3)

    rel0, rel1, aux = _dense_call(
        questions, word_emb,
        wih_cat, bih_cat, whh_cat, bhh_cat,
        W_step0, b_step0.reshape(1, -1), W_step1, b_step1.reshape(1, -1),
        W_rel, b_rel.reshape(1, -1), W_hop, b_hop.reshape(1, -1))

    es_t = e_s.T
    rel0_t = rel0.T
    rel1_t = rel1.T
    aux_t = aux.T

    npad = EDGE_PAD - T_EDGES
    subj_p = jnp.concatenate(
        [subj_idx, jnp.zeros((npad,), jnp.int32)]).reshape(ROWS128, 128)
    reli_p = jnp.concatenate(
        [rel_idx, jnp.zeros((npad,), jnp.int32)]).reshape(ROWS128, 128)
    dump = NUM_ENT + jnp.arange(npad, dtype=jnp.int32) % (ENTP - NUM_ENT)
    obj_p = jnp.concatenate([obj_idx, dump]).reshape(ROWS128, 128)

    zrows = jnp.zeros((ZR, 16), dtype=f32)

    part0 = _hop_call(es_t, rel0_t, subj_p, reli_p, obj_p, zrows)
    e1 = _combine0_call(part0)
    part1 = _hop_call(e1, rel1_t, subj_p, reli_p, obj_p, zrows)
    return _final_call(part1, e1, es_t, aux_t)


# speedup vs baseline: 1.0737x; 1.0158x over previous
import functools

import jax
import jax.numpy as jnp
from jax import lax
from jax.experimental import pallas as pl
from jax.experimental.pallas import tpu as pltpu
from jax.experimental.pallas import tpu_sc as plsc

NUM_ENT = 50000
NUM_REL = 512
NUM_STEPS = 2
BSZ = 16
SEQ = 32
T_EDGES = 800000
DIM_HIDDEN = 768
H = 384
G3 = 3 * H

NC = 2
NS = 16
NW = NC * NS

ENTP = 50176
ZR = ENTP // NS
CROWS = ENTP // NW

EDGE_PAD = 819200
ROWS128 = EDGE_PAD // 128
ROWS_PER_W = ROWS128 // NW
CHUNK_ROWS = 4
NCHUNK = ROWS_PER_W // CHUNK_ROWS


EREAL = T_EDGES // 128
EPADR = ROWS128 - EREAL


def _dense_body(q_ref, qs_ref, wemb_ref,
                wih_ref, bih_ref, whh_ref, bhh_ref,
                w0_ref, b0_ref, w1_ref, b1_ref, wr_ref, br_ref,
                wh_ref, bh_ref,
                rel0_ref, rel1_ref, aux_ref,
                x_ref, gi_ref, hs_ref, gsem):
    f32 = jnp.float32
    qz = (q_ref[...] == 0).astype(f32)
    lens = (SEQ - jnp.sum(qz, axis=1, keepdims=True))

    LAG = 64

    def _row_copy(r):
        b = jnp.remainder(r, BSZ)
        t = r // BSZ
        idx = qs_ref[b, t]
        return pltpu.make_async_copy(wemb_ref.at[idx], x_ref.at[r], gsem)

    def _fire(r, carry):
        _row_copy(r).start()

        @pl.when(r >= LAG)
        def _():
            _row_copy(r - LAG).wait()
        return carry

    lax.fori_loop(0, SEQ * BSZ, _fire, 0)

    def _drain(r, carry):
        _row_copy(r).wait()
        return carry

    lax.fori_loop(SEQ * BSZ - LAG, SEQ * BSZ, _drain, 0)

    gi_ref[...] = (
        jnp.dot(x_ref[...], wih_ref[...], preferred_element_type=f32)
        + bih_ref[...]
    )

    def gru_dir(gi, h, gh):
        i_r = gi[:, 0:H]
        i_z = gi[:, H:2 * H]
        i_n = gi[:, 2 * H:3 * H]
        h_r = gh[:, 0:H]
        h_z = gh[:, H:2 * H]
        h_n = gh[:, 2 * H:3 * H]
        r = jax.nn.sigmoid(i_r + h_r)
        z = jax.nn.sigmoid(i_z + h_z)
        n = jnp.tanh(i_n + r * h_n)
        return (1.0 - z) * n + z * h

    def step(t, h_cat):
        h_f = h_cat[:, 0:H]
        h_b = h_cat[:, H:2 * H]
        gh = jnp.dot(h_cat, whh_ref[...], preferred_element_type=f32) + bhh_ref[...]
        gi_f = gi_ref[pl.ds(t * BSZ, BSZ), 0:G3]
        tb = SEQ - 1 - t
        gi_b = gi_ref[pl.ds(tb * BSZ, BSZ), G3:2 * G3]
        hf_new = gru_dir(gi_f, h_f, gh[:, 0:G3])
        hb_new = gru_dir(gi_b, h_b, gh[:, G3:2 * G3])
        mt_f = (t.astype(f32) < lens).astype(f32)
        mt_b = ((SEQ - 1 - t).astype(f32) < lens).astype(f32)
        h_f2 = mt_f * hf_new + (1.0 - mt_f) * h_f
        h_b2 = mt_b * hb_new + (1.0 - mt_b) * h_b
        hs_ref[pl.ds(t * BSZ, BSZ), 0:H] = h_f2
        hs_ref[pl.ds(tb * BSZ, BSZ), H:2 * H] = h_b2
        return jnp.concatenate([h_f2, h_b2], axis=1)

    h0 = jnp.zeros((BSZ, 2 * H), dtype=f32)
    hT = lax.fori_loop(0, SEQ, step, h0)
    q_emb = hT

    step_w = [w0_ref, w1_ref]
    step_b = [b0_ref, b1_ref]
    rel_refs = [rel0_ref, rel1_ref]
    ams = []
    for t in range(NUM_STEPS):
        cq = jnp.tanh(
            jnp.dot(q_emb, step_w[t][...], preferred_element_type=f32)
            + step_b[t][...]
        )
        cols = []
        for s in range(SEQ):
            blk = hs_ref[pl.ds(s * BSZ, BSZ), :]
            cols.append(jnp.sum(cq * blk, axis=1, keepdims=True))
        logits = jnp.concatenate(cols, axis=1)
        mx = jnp.max(logits, axis=1, keepdims=True)
        ex = jnp.exp(logits - mx)
        dist = ex / jnp.sum(ex, axis=1, keepdims=True)
        ctx = jnp.zeros((BSZ, 2 * H), dtype=f32)
        for s in range(SEQ):
            ctx = ctx + dist[:, s:s + 1] * hs_ref[pl.ds(s * BSZ, BSZ), :]
        rl = jnp.dot(ctx, wr_ref[...], preferred_element_type=f32) + br_ref[...]
        rmx = jnp.max(rl, axis=1, keepdims=True)
        rex = jnp.exp(rl - rmx)
        rel_refs[t][...] = rex / jnp.sum(rex, axis=1, keepdims=True)
        ii = lax.broadcasted_iota(jnp.int32, (BSZ, NUM_REL), 1)
        cand = jnp.where(rl >= rmx, ii, NUM_REL)
        ams.append(jnp.min(cand, axis=1, keepdims=True))

    prev_rel, curr_rel = ams[0], ams[1]
    cond = ((jnp.abs(prev_rel - curr_rel) == 1)
            & (jnp.remainder(jnp.minimum(prev_rel, curr_rel), 2) == 0))
    hop_logit = jnp.dot(q_emb, wh_ref[...], preferred_element_type=f32) + bh_ref[...]
    hmx = jnp.max(hop_logit, axis=1, keepdims=True)
    hex_ = jnp.exp(hop_logit - hmx)
    attn = hex_ / jnp.sum(hex_, axis=1, keepdims=True)
    a0 = attn[:, 0:1]
    a1 = attn[:, 1:2]
    m3 = (a1 > a0).astype(f32)
    aux_ref[:, 0:1] = cond.astype(f32)
    aux_ref[:, 1:2] = a0
    aux_ref[:, 2:3] = a1
    aux_ref[:, 3:4] = m3


def _dense_call(questions, word_emb,
                wih_cat, bih_cat, whh_cat, bhh_cat,
                w0, b0, w1, b1, wr, br, wh, bh):
    f32 = jnp.float32
    nin = 15
    specs = [pl.BlockSpec(memory_space=pltpu.VMEM) for _ in range(nin)]
    specs[0] = pl.BlockSpec(memory_space=pltpu.VMEM)
    specs[1] = pl.BlockSpec(memory_space=pltpu.SMEM)
    specs[2] = pl.BlockSpec(memory_space=pl.ANY)
    return pl.pallas_call(
        _dense_body,
        out_shape=(
            jax.ShapeDtypeStruct((BSZ, NUM_REL), f32),
            jax.ShapeDtypeStruct((BSZ, NUM_REL), f32),
            jax.ShapeDtypeStruct((BSZ, 4), f32),
        ),
        in_specs=specs,
        scratch_shapes=[
            pltpu.VMEM((SEQ * BSZ, 300), f32),
            pltpu.VMEM((SEQ * BSZ, 2 * G3), f32),
            pltpu.VMEM((SEQ * BSZ, 2 * H), f32),
            pltpu.SemaphoreType.DMA,
        ],
    )(questions, questions, word_emb,
      wih_cat, bih_cat, whh_cat, bhh_cat,
      w0, b0, w1, b1, wr, br, wh, bh)


_SC_MESH = plsc.VectorSubcoreMesh(
    core_axis_name="c", subcore_axis_name="s", num_cores=NC, num_subcores=NS)


def _hop_body(ent, relt, subj, reli, obj, zrows, out,
              subj_v, rel_v, obj_v, erow, rrow, sem_g, sem_s, acc):
    c = lax.axis_index("c")
    s = lax.axis_index("s")
    wid = s * NC + c

    pltpu.sync_copy(zrows, acc.at[pl.ds(s * ZR, ZR)])
    plsc.subcore_barrier()

    def fire(kk, b):
        row0 = wid * ROWS_PER_W + kk * CHUNK_ROWS
        pltpu.sync_copy(subj.at[pl.ds(row0, CHUNK_ROWS)], subj_v.at[b])
        pltpu.sync_copy(reli.at[pl.ds(row0, CHUNK_ROWS)], rel_v.at[b])
        pltpu.sync_copy(obj.at[pl.ds(row0, CHUNK_ROWS)], obj_v.at[b])
        for r in range(CHUNK_ROWS):
            pltpu.async_copy(ent.at[subj_v.at[b, r]],
                             erow.at[b, pl.ds(r * 128, 128)], sem_g)
            pltpu.async_copy(relt.at[rel_v.at[b, r]],
                             rrow.at[b, pl.ds(r * 128, 128)], sem_g)

    def wait_gathers(b):
        for r in range(CHUNK_ROWS):
            pltpu.make_async_copy(ent.at[subj_v.at[b, r]],
                                  erow.at[b, pl.ds(r * 128, 128)], sem_g).wait()
            pltpu.make_async_copy(relt.at[rel_v.at[b, r]],
                                  rrow.at[b, pl.ds(r * 128, 128)], sem_g).wait()

    def scatter(b):
        for r in range(CHUNK_ROWS):
            pltpu.async_copy(rrow.at[b, pl.ds(r * 128, 128)],
                             acc.at[obj_v.at[b, r]], sem_s, add=True)

    def drain_scatter(b):
        for r in range(CHUNK_ROWS):
            pltpu.make_async_copy(rrow.at[b, pl.ds(r * 128, 128)],
                                  acc.at[obj_v.at[b, r]], sem_s).wait()

    def mul(b):
        @pl.loop(0, CHUNK_ROWS * 128, unroll=16)
        def _mul(j):
            rrow[b, j, :] = rrow[b, j, :] * erow[b, j, :]

    fire(0, 0)
    fire(1, 1)

    @pl.loop(0, NCHUNK - 2, step=4)
    def _quad(k):
        for j in range(4):
            kk = k + j
            b = j
            n2 = (j + 2) % 4

            @pl.when(kk >= 2)
            def _():
                drain_scatter(n2)

            fire(kk + 2, n2)
            wait_gathers(b)
            mul(b)
            scatter(b)

    drain_scatter(2)
    wait_gathers(0)
    mul(0)
    scatter(0)
    drain_scatter(3)
    wait_gathers(1)
    mul(1)
    scatter(1)
    drain_scatter(0)
    drain_scatter(1)

    plsc.subcore_barrier()
    pltpu.sync_copy(acc.at[pl.ds(s * ZR, ZR)],
                    out.at[pl.ds(c * ENTP + s * ZR, ZR)])


_hop_call = functools.partial(
    pl.kernel,
    out_type=jax.ShapeDtypeStruct((NC * ENTP, 16), jnp.float32),
    mesh=_SC_MESH,
    compiler_params=pltpu.CompilerParams(
        use_tc_tiling_on_sc=False, needs_layout_passes=False),
    scratch_types=[
        pltpu.VMEM((4, CHUNK_ROWS, 128), jnp.int32),
        pltpu.VMEM((4, CHUNK_ROWS, 128), jnp.int32),
        pltpu.VMEM((4, CHUNK_ROWS, 128), jnp.int32),
        pltpu.VMEM((4, CHUNK_ROWS * 128, 16), jnp.float32),
        pltpu.VMEM((4, CHUNK_ROWS * 128, 16), jnp.float32),
        pltpu.SemaphoreType.DMA,
        pltpu.SemaphoreType.DMA,
        pltpu.VMEM_SHARED((ENTP, 16), jnp.float32),
    ],
)(_hop_body)


def _combine0_body(part, out, p0, p1, sem):
    c = lax.axis_index("c")
    s = lax.axis_index("s")
    wid = s * NC + c
    r0 = wid * CROWS
    cp0 = pltpu.make_async_copy(part.at[pl.ds(r0, CROWS)], p0, sem)
    cp1 = pltpu.make_async_copy(part.at[pl.ds(ENTP + r0, CROWS)], p1, sem)
    cp0.start(); cp1.start(); cp0.wait(); cp1.wait()

    @pl.loop(0, CROWS, unroll=4)
    def _row(j):
        v = p0[j, :] + p1[j, :]
        p0[j, :] = v / jnp.maximum(v, 1.0)

    pltpu.sync_copy(p0, out.at[pl.ds(r0, CROWS)])


_combine0_call = functools.partial(
    pl.kernel,
    out_type=jax.ShapeDtypeStruct((ENTP, 16), jnp.float32),
    mesh=_SC_MESH,
    compiler_params=pltpu.CompilerParams(
        use_tc_tiling_on_sc=False, needs_layout_passes=False),
    scratch_types=[
        pltpu.VMEM((CROWS, 16), jnp.float32),
        pltpu.VMEM((CROWS, 16), jnp.float32),
        pltpu.SemaphoreType.DMA,
    ],
)(_combine0_body)


def _final_body(part, e1, est, aux, out, p0, p1, e1v, esv, auxv, outt, sem):
    c = lax.axis_index("c")
    s = lax.axis_index("s")
    wid = s * NC + c
    r0 = wid * CROWS
    cp0 = pltpu.make_async_copy(part.at[pl.ds(r0, CROWS)], p0, sem)
    cp1 = pltpu.make_async_copy(part.at[pl.ds(ENTP + r0, CROWS)], p1, sem)
    cp2 = pltpu.make_async_copy(e1.at[pl.ds(r0, CROWS)], e1v, sem)
    cp0.start(); cp1.start(); cp2.start()
    esr = NUM_ENT - (NW - 1) * CROWS

    @pl.when(wid < NW - 1)
    def _():
        cp = pltpu.make_async_copy(est.at[pl.ds(r0, CROWS)], esv, sem)
        cp.start(); cp.wait()

    @pl.when(wid == NW - 1)
    def _():
        cp = pltpu.make_async_copy(est.at[pl.ds(r0, esr)],
                                   esv.at[pl.ds(0, esr)], sem)
        cp.start(); cp.wait()

    pltpu.sync_copy(aux, auxv)
    cp0.wait(); cp1.wait(); cp2.wait()
    cond = auxv[0, :]
    a0 = auxv[1, :]
    a1 = auxv[2, :]
    m3 = auxv[3, :]

    @pl.loop(0, CROWS, unroll=4)
    def _row(j):
        v = p0[j, :] + p1[j, :]
        v = v / jnp.maximum(v, 1.0)
        es_row = esv[j, :]
        ent_m = cond * jnp.where(es_row > 0.9, 1.0, 0.0)
        v = (1.0 - ent_m) * v
        o = a0 * e1v[j, :] + a1 * v
        p0[j, :] = (1.0 - m3 * es_row) * o

    lanes = lax.iota(jnp.int32, 16)

    @pl.loop(0, CROWS // 16)
    def _tr(jv):
        rows = jv * 16 + lanes
        for b in range(BSZ):
            col = jnp.full((16,), b, jnp.int32)
            outt[b, pl.ds(jv * 16, 16)] = plsc.load_gather(p0, [rows, col])

    last = NUM_ENT - (NW - 1) * CROWS

    @pl.when(wid < NW - 1)
    def _():
        for b in range(BSZ):
            pltpu.sync_copy(outt.at[b], out.at[b, pl.ds(r0, CROWS)])

    @pl.when(wid == NW - 1)
    def _():
        for b in range(BSZ):
            pltpu.sync_copy(outt.at[b, pl.ds(0, last)],
                            out.at[b, pl.ds(r0, last)])


_final_call = functools.partial(
    pl.kernel,
    out_type=jax.ShapeDtypeStruct((BSZ, NUM_ENT), jnp.float32),
    mesh=_SC_MESH,
    compiler_params=pltpu.CompilerParams(
        use_tc_tiling_on_sc=False, needs_layout_passes=False),
    scratch_types=[
        pltpu.VMEM((CROWS, 16), jnp.float32),
        pltpu.VMEM((CROWS, 16), jnp.float32),
        pltpu.VMEM((CROWS, 16), jnp.float32),
        pltpu.VMEM((CROWS, 16), jnp.float32),
        pltpu.VMEM((4, 16), jnp.float32),
        pltpu.VMEM((BSZ, CROWS), jnp.float32),
        pltpu.SemaphoreType.DMA,
    ],
)(_final_body)


def kernel(questions, e_s, subj_idx, rel_idx, obj_idx, word_emb,
           Wf_ih, Wf_hh, bf_ih, bf_hh, Wb_ih, Wb_hh, bb_ih, bb_hh,
           W_step0, b_step0, W_step1, b_step1, W_rel, b_rel, W_hop, b_hop):
    f32 = jnp.float32

    wih_cat = jnp.concatenate([Wf_ih, Wb_ih], axis=1)
    bih_cat = jnp.concatenate([bf_ih, bb_ih]).reshape(1, 2 * G3)
    zhh = jnp.zeros((H, G3), dtype=f32)
    whh_cat = jnp.concatenate([
        jnp.concatenate([Wf_hh, zhh], axis=1),
        jnp.concatenate([zhh, Wb_hh], axis=1),
    ], axis=0)
    bhh_cat = jnp.concatenate([bf_hh, bb_hh]).reshape(1, 2 * G3)

    rel0, rel1, aux = _dense_call(
        questions, word_emb,
        wih_cat, bih_cat, whh_cat, bhh_cat,
        W_step0, b_step0.reshape(1, -1), W_step1, b_step1.reshape(1, -1),
        W_rel, b_rel.reshape(1, -1), W_hop, b_hop.reshape(1, -1))

    es_t = e_s.T
    rel0_t = rel0.T
    rel1_t = rel1.T
    aux_t = aux.T

    npad = EDGE_PAD - T_EDGES
    subj_p = jnp.concatenate(
        [subj_idx, jnp.zeros((npad,), jnp.int32)]).reshape(ROWS128, 128)
    reli_p = jnp.concatenate(
        [rel_idx, jnp.zeros((npad,), jnp.int32)]).reshape(ROWS128, 128)
    dump = NUM_ENT + jnp.arange(npad, dtype=jnp.int32) % (ENTP - NUM_ENT)
    obj_p = jnp.concatenate([obj_idx, dump]).reshape(ROWS128, 128)

    zrows = jnp.zeros((ZR, 16), dtype=f32)

    part0 = _hop_call(es_t, rel0_t, subj_p, reli_p, obj_p, zrows)
    e1 = _combine0_call(part0)
    part1 = _hop_call(e1, rel1_t, subj_p, reli_p, obj_p, zrows)
    return _final_call(part1, e1, es_t, aux_t)
